# baseline (speedup 1.0000x reference)
import functools

import jax
import jax.numpy as jnp
from jax import lax
from jax.experimental import pallas as pl
from jax.experimental.pallas import tpu as pltpu
from jax.experimental.pallas import tpu_sc as plsc

NUM_EMB = 8192
DIM = 256
BATCH = 16
SEQ = 1024
N_TOKENS = BATCH * SEQ
BLOCKS = ((0, 2736), (2736, 2736), (5472, 2720))


ROWS = 16


def _argmin_body(zn_ref, cn_ref, z_ref, cb_ref, idx_ref, loss_ref, s_scr):
    z = z_ref[0]
    zn = zn_ref[0]

    accv = None
    acci = None
    chosen = None
    for start, size in BLOCKS:
        cb = cb_ref[start:start + size, :]
        s_scr[0:size, :] = lax.dot_general(
            cb, z, (((1,), (0,)), ((), ())),
            preferred_element_type=jnp.float32)

        riota = lax.broadcasted_iota(jnp.int32, (ROWS, 1), 0)

        def step(i, carry):
            runv, runi = carry
            sv = s_scr[pl.ds(i * ROWS, ROWS), :]
            cnv = cn_ref[pl.ds(start + i * ROWS, ROWS), :]
            d = (zn + cnv) - 2.0 * sv
            rows = start + i * ROWS + riota
            lt = d < runv
            return (jnp.where(lt, d, runv), jnp.where(lt, rows, runi))

        runv0 = jnp.full((ROWS, SEQ), jnp.inf, jnp.float32)
        runi0 = jnp.zeros((ROWS, SEQ), jnp.int32)
        runv, runi = lax.fori_loop(0, size // ROWS, step, (runv0, runi0))

        bmin = jnp.min(runv, axis=0, keepdims=True)
        bidx = jnp.min(jnp.where(runv == bmin, runi, jnp.int32(2**30)),
                       axis=0, keepdims=True)
        if accv is None:
            chosen, acci, accv = bmin, bidx, bmin
        else:
            keep = accv <= bmin
            chosen = jnp.where(keep, chosen, bmin)
            acci = jnp.where(keep, acci, bidx)
            accv = jnp.where(keep, accv, bmin)
        accv = accv.astype(jnp.bfloat16).astype(jnp.float32)

    idx_ref[0] = acci
    loss_ref[0, 0, 0] = jnp.sum(chosen)


def _argmin_call(zn, cn, z, codebook):
    return pl.pallas_call(
        _argmin_body,
        grid=(BATCH,),
        in_specs=[
            pl.BlockSpec((1, 1, SEQ), lambda b: (b, 0, 0)),
            pl.BlockSpec((NUM_EMB, 1), lambda b: (0, 0)),
            pl.BlockSpec((1, DIM, SEQ), lambda b: (b, 0, 0)),
            pl.BlockSpec((NUM_EMB, DIM), lambda b: (0, 0)),
        ],
        out_specs=[
            pl.BlockSpec((1, 1, SEQ), lambda b: (b, 0, 0)),
            pl.BlockSpec((1, 1, 1), lambda b: (b, 0, 0),
                         memory_space=pltpu.SMEM),
        ],
        out_shape=[
            jax.ShapeDtypeStruct((BATCH, 1, SEQ), jnp.int32),
            jax.ShapeDtypeStruct((BATCH, 1, 1), jnp.float32),
        ],
        scratch_shapes=[
            pltpu.VMEM((BLOCKS[0][1], SEQ), jnp.float32),
        ],
        compiler_params=pltpu.CompilerParams(
            dimension_semantics=("parallel",)),
    )(zn, cn, z, codebook)


def _gather_call(codebook, idx_flat):
    info = plsc.get_sparse_core_info()
    nw = info.num_cores * info.num_subcores
    chunk = 128
    per_w = N_TOKENS // nw
    n_chunks = per_w // chunk
    mesh = plsc.VectorSubcoreMesh(core_axis_name="c", subcore_axis_name="s")

    @functools.partial(
        pl.kernel, mesh=mesh,
        out_type=jax.ShapeDtypeStruct((N_TOKENS, DIM), jnp.float32),
        scratch_types=[
            pltpu.VMEM((chunk,), jnp.int32),
            pltpu.VMEM((chunk, DIM), jnp.float32),
            pltpu.SemaphoreType.DMA,
        ],
    )
    def k(table_hbm, idx_hbm, out_hbm, idx_v, rows_v, sem):
        wid = lax.axis_index("s") * info.num_cores + lax.axis_index("c")
        base = wid * per_w
        for c in range(n_chunks):
            off = base + c * chunk
            pltpu.sync_copy(idx_hbm.at[pl.ds(off, chunk)], idx_v)
            pltpu.async_copy(table_hbm.at[idx_v], rows_v, sem).wait()
            pltpu.sync_copy(rows_v, out_hbm.at[pl.ds(off, chunk)])

    return k(codebook, idx_flat)


def _transpose_body(q_ref, o_ref):
    o_ref[0] = q_ref[0].T


def _transpose_call(q_flat):
    return pl.pallas_call(
        _transpose_body,
        grid=(BATCH,),
        in_specs=[pl.BlockSpec((1, SEQ, DIM), lambda b: (b, 0, 0))],
        out_specs=pl.BlockSpec((1, DIM, SEQ), lambda b: (b, 0, 0)),
        out_shape=jax.ShapeDtypeStruct((BATCH, DIM, SEQ), jnp.float32),
        compiler_params=pltpu.CompilerParams(
            dimension_semantics=("parallel",)),
    )(q_flat.reshape(BATCH, SEQ, DIM))


def kernel(z, codebook):
    z_flat = jnp.transpose(z, (0, 2, 1)).reshape(-1, DIM)
    zn = jnp.sum(z_flat ** 2, axis=1).reshape(BATCH, 1, SEQ)
    cn = jnp.sum(codebook ** 2, axis=1, keepdims=True)
    idx3, loss_parts = _argmin_call(zn, cn, z, codebook)
    indices = idx3.reshape(BATCH, SEQ)
    q_flat = _gather_call(codebook, indices.reshape(-1))
    quantized_out = _transpose_call(q_flat)
    vq_loss = jnp.sum(loss_parts) * (1.25 / (N_TOKENS * DIM))
    return (vq_loss, quantized_out, indices)


# speedup vs baseline: 4.4743x; 4.4743x over previous
import functools

import jax
import jax.numpy as jnp
from jax import lax
from jax.experimental import pallas as pl
from jax.experimental.pallas import tpu as pltpu
from jax.experimental.pallas import tpu_sc as plsc

NUM_EMB = 8192
DIM = 256
BATCH = 16
SEQ = 1024
N_TOKENS = BATCH * SEQ
BLOCKS = ((0, 2736), (2736, 2736), (5472, 2720))


ROWS = 16


def _argmin_body(zn_ref, cn_ref, z_ref, cb_ref, idx_ref, loss_ref, s_scr):
    z = z_ref[0]
    zn = zn_ref[0]

    accv = None
    acci = None
    chosen = None
    for start, size in BLOCKS:
        cb = cb_ref[start:start + size, :]
        s_scr[0:size, :] = lax.dot_general(
            cb, z, (((1,), (0,)), ((), ())),
            preferred_element_type=jnp.float32)

        riota = lax.broadcasted_iota(jnp.int32, (ROWS, 1), 0)
        runv = jnp.full((ROWS, SEQ), jnp.inf, jnp.float32)
        runi = jnp.zeros((ROWS, SEQ), jnp.int32)
        for c in range(size // ROWS):
            sv = s_scr[c * ROWS:(c + 1) * ROWS, :]
            cnv = cn_ref[start + c * ROWS:start + (c + 1) * ROWS, :]
            d = (zn + cnv) - 2.0 * sv
            rows = start + c * ROWS + riota
            lt = d < runv
            runv = jnp.where(lt, d, runv)
            runi = jnp.where(lt, rows, runi)

        bmin = jnp.min(runv, axis=0, keepdims=True)
        bidx = jnp.min(jnp.where(runv == bmin, runi, jnp.int32(2**30)),
                       axis=0, keepdims=True)
        if accv is None:
            chosen, acci, accv = bmin, bidx, bmin
        else:
            keep = accv <= bmin
            chosen = jnp.where(keep, chosen, bmin)
            acci = jnp.where(keep, acci, bidx)
            accv = jnp.where(keep, accv, bmin)
        accv = accv.astype(jnp.bfloat16).astype(jnp.float32)

    idx_ref[0] = acci
    loss_ref[0, 0, 0] = jnp.sum(chosen)


def _argmin_call(zn, cn, z, codebook):
    return pl.pallas_call(
        _argmin_body,
        grid=(BATCH,),
        in_specs=[
            pl.BlockSpec((1, 1, SEQ), lambda b: (b, 0, 0)),
            pl.BlockSpec((NUM_EMB, 1), lambda b: (0, 0)),
            pl.BlockSpec((1, DIM, SEQ), lambda b: (b, 0, 0)),
            pl.BlockSpec((NUM_EMB, DIM), lambda b: (0, 0)),
        ],
        out_specs=[
            pl.BlockSpec((1, 1, SEQ), lambda b: (b, 0, 0)),
            pl.BlockSpec((1, 1, 1), lambda b: (b, 0, 0),
                         memory_space=pltpu.SMEM),
        ],
        out_shape=[
            jax.ShapeDtypeStruct((BATCH, 1, SEQ), jnp.int32),
            jax.ShapeDtypeStruct((BATCH, 1, 1), jnp.float32),
        ],
        scratch_shapes=[
            pltpu.VMEM((BLOCKS[0][1], SEQ), jnp.float32),
        ],
        compiler_params=pltpu.CompilerParams(
            dimension_semantics=("parallel",)),
    )(zn, cn, z, codebook)


def _gather_call(codebook, idx_flat):
    info = plsc.get_sparse_core_info()
    nw = info.num_cores * info.num_subcores
    chunk = 128
    per_w = N_TOKENS // nw
    n_chunks = per_w // chunk
    mesh = plsc.VectorSubcoreMesh(core_axis_name="c", subcore_axis_name="s")

    @functools.partial(
        pl.kernel, mesh=mesh,
        out_type=jax.ShapeDtypeStruct((N_TOKENS, DIM), jnp.float32),
        scratch_types=[
            pltpu.VMEM((chunk,), jnp.int32),
            pltpu.VMEM((chunk, DIM), jnp.float32),
            pltpu.SemaphoreType.DMA,
        ],
    )
    def k(table_hbm, idx_hbm, out_hbm, idx_v, rows_v, sem):
        wid = lax.axis_index("s") * info.num_cores + lax.axis_index("c")
        base = wid * per_w
        for c in range(n_chunks):
            off = base + c * chunk
            pltpu.sync_copy(idx_hbm.at[pl.ds(off, chunk)], idx_v)
            pltpu.async_copy(table_hbm.at[idx_v], rows_v, sem).wait()
            pltpu.sync_copy(rows_v, out_hbm.at[pl.ds(off, chunk)])

    return k(codebook, idx_flat)


def _transpose_body(q_ref, o_ref):
    o_ref[0] = q_ref[0].T


def _transpose_call(q_flat):
    return pl.pallas_call(
        _transpose_body,
        grid=(BATCH,),
        in_specs=[pl.BlockSpec((1, SEQ, DIM), lambda b: (b, 0, 0))],
        out_specs=pl.BlockSpec((1, DIM, SEQ), lambda b: (b, 0, 0)),
        out_shape=jax.ShapeDtypeStruct((BATCH, DIM, SEQ), jnp.float32),
        compiler_params=pltpu.CompilerParams(
            dimension_semantics=("parallel",)),
    )(q_flat.reshape(BATCH, SEQ, DIM))


def kernel(z, codebook):
    z_flat = jnp.transpose(z, (0, 2, 1)).reshape(-1, DIM)
    zn = jnp.sum(z_flat ** 2, axis=1).reshape(BATCH, 1, SEQ)
    cn = jnp.sum(codebook ** 2, axis=1, keepdims=True)
    idx3, loss_parts = _argmin_call(zn, cn, z, codebook)
    indices = idx3.reshape(BATCH, SEQ)
    q_flat = _gather_call(codebook, indices.reshape(-1))
    quantized_out = _transpose_call(q_flat)
    vq_loss = jnp.sum(loss_parts) * (1.25 / (N_TOKENS * DIM))
    return (vq_loss, quantized_out, indices)


# speedup vs baseline: 4.6557x; 1.0406x over previous
import functools

import jax
import jax.numpy as jnp
from jax import lax
from jax.experimental import pallas as pl
from jax.experimental.pallas import tpu as pltpu
from jax.experimental.pallas import tpu_sc as plsc

NUM_EMB = 8192
DIM = 256
BATCH = 16
SEQ = 1024
N_TOKENS = BATCH * SEQ
BLOCKS = ((0, 2736), (2736, 2736), (5472, 2720))


ROWS = 16


def _argmin_body(zn_ref, cn_ref, z_ref, cb_ref, idx_ref, loss_ref, s_scr):
    z = z_ref[0]
    zn = zn_ref[0]

    accv = None
    acci = None
    chosen = None
    for start, size in BLOCKS:
        cb2 = cb_ref[start:start + size, :]
        s_scr[0:size, :] = lax.dot_general(
            cb2, z, (((1,), (0,)), ((), ())),
            preferred_element_type=jnp.float32)

        riota = lax.broadcasted_iota(jnp.int32, (ROWS, 1), 0)
        runv = jnp.full((ROWS, SEQ), jnp.inf, jnp.float32)
        runi = jnp.zeros((ROWS, SEQ), jnp.int32)
        for c in range(size // ROWS):
            sv = s_scr[c * ROWS:(c + 1) * ROWS, :]
            cnv = cn_ref[start + c * ROWS:start + (c + 1) * ROWS, :]
            d = (zn + cnv) - sv
            rows = start + c * ROWS + riota
            lt = d < runv
            runv = jnp.where(lt, d, runv)
            runi = jnp.where(lt, rows, runi)

        bmin = jnp.min(runv, axis=0, keepdims=True)
        bidx = jnp.min(jnp.where(runv == bmin, runi, jnp.int32(2**30)),
                       axis=0, keepdims=True)
        if accv is None:
            chosen, acci, accv = bmin, bidx, bmin
        else:
            keep = accv <= bmin
            chosen = jnp.where(keep, chosen, bmin)
            acci = jnp.where(keep, acci, bidx)
            accv = jnp.where(keep, accv, bmin)
        accv = accv.astype(jnp.bfloat16).astype(jnp.float32)

    idx_ref[0] = acci
    loss_ref[0, 0, 0] = jnp.sum(chosen)


def _argmin_call(zn, cn, z, codebook):
    return pl.pallas_call(
        _argmin_body,
        grid=(BATCH,),
        in_specs=[
            pl.BlockSpec((1, 1, SEQ), lambda b: (b, 0, 0)),
            pl.BlockSpec((NUM_EMB, 1), lambda b: (0, 0)),
            pl.BlockSpec((1, DIM, SEQ), lambda b: (b, 0, 0)),
            pl.BlockSpec((NUM_EMB, DIM), lambda b: (0, 0)),
        ],
        out_specs=[
            pl.BlockSpec((1, 1, SEQ), lambda b: (b, 0, 0)),
            pl.BlockSpec((1, 1, 1), lambda b: (b, 0, 0),
                         memory_space=pltpu.SMEM),
        ],
        out_shape=[
            jax.ShapeDtypeStruct((BATCH, 1, SEQ), jnp.int32),
            jax.ShapeDtypeStruct((BATCH, 1, 1), jnp.float32),
        ],
        scratch_shapes=[
            pltpu.VMEM((BLOCKS[0][1], SEQ), jnp.float32),
        ],
        compiler_params=pltpu.CompilerParams(
            dimension_semantics=("parallel",)),
    )(zn, cn, z, codebook)


def _gather_call(codebook, idx_flat):
    info = plsc.get_sparse_core_info()
    nw = info.num_cores * info.num_subcores
    chunk = 128
    per_w = N_TOKENS // nw
    n_chunks = per_w // chunk
    mesh = plsc.VectorSubcoreMesh(core_axis_name="c", subcore_axis_name="s")

    @functools.partial(
        pl.kernel, mesh=mesh,
        out_type=jax.ShapeDtypeStruct((N_TOKENS, DIM), jnp.float32),
        scratch_types=[
            pltpu.VMEM((chunk,), jnp.int32),
            pltpu.VMEM((chunk, DIM), jnp.float32),
            pltpu.SemaphoreType.DMA,
        ],
    )
    def k(table_hbm, idx_hbm, out_hbm, idx_v, rows_v, sem):
        wid = lax.axis_index("s") * info.num_cores + lax.axis_index("c")
        base = wid * per_w
        for c in range(n_chunks):
            off = base + c * chunk
            pltpu.sync_copy(idx_hbm.at[pl.ds(off, chunk)], idx_v)
            pltpu.async_copy(table_hbm.at[idx_v], rows_v, sem).wait()
            pltpu.sync_copy(rows_v, out_hbm.at[pl.ds(off, chunk)])

    return k(codebook, idx_flat)


def _transpose_body(q_ref, o_ref):
    o_ref[0] = q_ref[0].T


def _transpose_call(q_flat):
    return pl.pallas_call(
        _transpose_body,
        grid=(BATCH,),
        in_specs=[pl.BlockSpec((1, SEQ, DIM), lambda b: (b, 0, 0))],
        out_specs=pl.BlockSpec((1, DIM, SEQ), lambda b: (b, 0, 0)),
        out_shape=jax.ShapeDtypeStruct((BATCH, DIM, SEQ), jnp.float32),
        compiler_params=pltpu.CompilerParams(
            dimension_semantics=("parallel",)),
    )(q_flat.reshape(BATCH, SEQ, DIM))


def kernel(z, codebook):
    z_flat = jnp.transpose(z, (0, 2, 1)).reshape(-1, DIM)
    zn = jnp.sum(z_flat ** 2, axis=1).reshape(BATCH, 1, SEQ)
    cn = jnp.sum(codebook ** 2, axis=1, keepdims=True)
    idx3, loss_parts = _argmin_call(zn, cn, z, codebook * 2.0)
    indices = idx3.reshape(BATCH, SEQ)
    q_flat = _gather_call(codebook, indices.reshape(-1))
    quantized_out = _transpose_call(q_flat)
    vq_loss = jnp.sum(loss_parts) * (1.25 / (N_TOKENS * DIM))
    return (vq_loss, quantized_out, indices)


# speedup vs baseline: 4.6750x; 1.0041x over previous
import functools

import jax
import jax.numpy as jnp
from jax import lax
from jax.experimental import pallas as pl
from jax.experimental.pallas import tpu as pltpu
from jax.experimental.pallas import tpu_sc as plsc

NUM_EMB = 8192
DIM = 256
BATCH = 16
SEQ = 1024
N_TOKENS = BATCH * SEQ
BLOCKS = ((0, 2736), (2736, 2736), (5472, 2720))


ROWS = 8


def _argmin_body(zn_ref, cn_ref, z_ref, cb_ref, idx_ref, loss_ref, s_scr):
    z = z_ref[0]
    zn = zn_ref[0]

    accv = None
    acci = None
    chosen = None
    for start, size in BLOCKS:
        cb2 = cb_ref[start:start + size, :]
        s_scr[0:size, :] = lax.dot_general(
            cb2, z, (((1,), (0,)), ((), ())),
            preferred_element_type=jnp.float32)

        riota = lax.broadcasted_iota(jnp.int32, (ROWS, 1), 0)
        runv = jnp.full((ROWS, SEQ), jnp.inf, jnp.float32)
        runi = jnp.zeros((ROWS, SEQ), jnp.int32)
        for c in range(size // ROWS):
            sv = s_scr[c * ROWS:(c + 1) * ROWS, :]
            cnv = cn_ref[start + c * ROWS:start + (c + 1) * ROWS, :]
            d = (zn + cnv) - sv
            rows = start + c * ROWS + riota
            lt = d < runv
            runv = jnp.where(lt, d, runv)
            runi = jnp.where(lt, rows, runi)

        bmin = jnp.min(runv, axis=0, keepdims=True)
        bidx = jnp.min(jnp.where(runv == bmin, runi, jnp.int32(2**30)),
                       axis=0, keepdims=True)
        if accv is None:
            chosen, acci, accv = bmin, bidx, bmin
        else:
            keep = accv <= bmin
            chosen = jnp.where(keep, chosen, bmin)
            acci = jnp.where(keep, acci, bidx)
            accv = jnp.where(keep, accv, bmin)
        accv = accv.astype(jnp.bfloat16).astype(jnp.float32)

    idx_ref[0] = acci
    loss_ref[0, 0, 0] = jnp.sum(chosen)


def _argmin_call(zn, cn, z, codebook):
    return pl.pallas_call(
        _argmin_body,
        grid=(BATCH,),
        in_specs=[
            pl.BlockSpec((1, 1, SEQ), lambda b: (b, 0, 0)),
            pl.BlockSpec((NUM_EMB, 1), lambda b: (0, 0)),
            pl.BlockSpec((1, DIM, SEQ), lambda b: (b, 0, 0)),
            pl.BlockSpec((NUM_EMB, DIM), lambda b: (0, 0)),
        ],
        out_specs=[
            pl.BlockSpec((1, 1, SEQ), lambda b: (b, 0, 0)),
            pl.BlockSpec((1, 1, 1), lambda b: (b, 0, 0),
                         memory_space=pltpu.SMEM),
        ],
        out_shape=[
            jax.ShapeDtypeStruct((BATCH, 1, SEQ), jnp.int32),
            jax.ShapeDtypeStruct((BATCH, 1, 1), jnp.float32),
        ],
        scratch_shapes=[
            pltpu.VMEM((BLOCKS[0][1], SEQ), jnp.float32),
        ],
        compiler_params=pltpu.CompilerParams(
            dimension_semantics=("parallel",)),
    )(zn, cn, z, codebook)


def _gather_call(codebook, idx_flat):
    info = plsc.get_sparse_core_info()
    nw = info.num_cores * info.num_subcores
    chunk = 128
    per_w = N_TOKENS // nw
    n_chunks = per_w // chunk
    mesh = plsc.VectorSubcoreMesh(core_axis_name="c", subcore_axis_name="s")

    @functools.partial(
        pl.kernel, mesh=mesh,
        out_type=jax.ShapeDtypeStruct((N_TOKENS, DIM), jnp.float32),
        scratch_types=[
            pltpu.VMEM((chunk,), jnp.int32),
            pltpu.VMEM((chunk, DIM), jnp.float32),
            pltpu.SemaphoreType.DMA,
        ],
    )
    def k(table_hbm, idx_hbm, out_hbm, idx_v, rows_v, sem):
        wid = lax.axis_index("s") * info.num_cores + lax.axis_index("c")
        base = wid * per_w
        for c in range(n_chunks):
            off = base + c * chunk
            pltpu.sync_copy(idx_hbm.at[pl.ds(off, chunk)], idx_v)
            pltpu.async_copy(table_hbm.at[idx_v], rows_v, sem).wait()
            pltpu.sync_copy(rows_v, out_hbm.at[pl.ds(off, chunk)])

    return k(codebook, idx_flat)


def _transpose_body(q_ref, o_ref):
    o_ref[0] = q_ref[0].T


def _transpose_call(q_flat):
    return pl.pallas_call(
        _transpose_body,
        grid=(BATCH,),
        in_specs=[pl.BlockSpec((1, SEQ, DIM), lambda b: (b, 0, 0))],
        out_specs=pl.BlockSpec((1, DIM, SEQ), lambda b: (b, 0, 0)),
        out_shape=jax.ShapeDtypeStruct((BATCH, DIM, SEQ), jnp.float32),
        compiler_params=pltpu.CompilerParams(
            dimension_semantics=("parallel",)),
    )(q_flat.reshape(BATCH, SEQ, DIM))


def kernel(z, codebook):
    z_flat = jnp.transpose(z, (0, 2, 1)).reshape(-1, DIM)
    zn = jnp.sum(z_flat ** 2, axis=1).reshape(BATCH, 1, SEQ)
    cn = jnp.sum(codebook ** 2, axis=1, keepdims=True)
    idx3, loss_parts = _argmin_call(zn, cn, z, codebook * 2.0)
    indices = idx3.reshape(BATCH, SEQ)
    q_flat = _gather_call(codebook, indices.reshape(-1))
    quantized_out = _transpose_call(q_flat)
    vq_loss = jnp.sum(loss_parts) * (1.25 / (N_TOKENS * DIM))
    return (vq_loss, quantized_out, indices)
